# baseline (speedup 1.0000x reference)
import functools

import jax
import jax.numpy as jnp
from jax import lax
from jax.experimental import pallas as pl
from jax.experimental.pallas import tpu as pltpu
from jax.experimental.pallas import tpu_sc as plsc

_NC = 2
_NS = 16
_NW = _NC * _NS
_CHUNK = 128
_NBUF = 5
_CB = 256


def _detile_table(w_t, v, d):
    n_full = v // _CB
    tail = v - n_full * _CB
    n_g = -(-n_full // _NW)
    d_groups = d // 16

    mesh = plsc.VectorSubcoreMesh(core_axis_name="c", subcore_axis_name="s")

    @functools.partial(
        pl.kernel,
        out_type=jax.ShapeDtypeStruct((v * d,), jnp.float32),
        mesh=mesh,
        scratch_types=[
            pltpu.VMEM((d, _CB), jnp.float32),
            pltpu.VMEM((d, _CB), jnp.float32),
            pltpu.VMEM((_CB * d,), jnp.float32),
            pltpu.VMEM((_CB * d,), jnp.float32),
            pltpu.VMEM((d, tail if tail else 1), jnp.float32),
        ] + [pltpu.SemaphoreType.DMA] * 4,
        compiler_params=pltpu.CompilerParams(
            use_tc_tiling_on_sc=True, needs_layout_passes=False
        ),
    )
    def detile_kernel(
        wt_hbm, stage_hbm, tin0, tin1, tout0, tout1, ttail, si0, si1, so0, so1
    ):
        tin = (tin0, tin1)
        tout = (tout0, tout1)
        wid = lax.axis_index("s") * _NC + lax.axis_index("c")
        sin = (si0, si1)
        sout = (so0, so1)
        lane_stride = jnp.arange(16, dtype=jnp.int32) * d
        c_iota = [
            jnp.arange(16, dtype=jnp.int32) + cg * 16 for cg in range(_CB // 16)
        ]

        def fire_in(j, slot):
            pltpu.async_copy(
                wt_hbm.at[:, pl.ds(j * _CB, _CB)], tin[slot], sin[slot]
            )

        def transpose_block(slot):
            dst = tout[slot]

            @plsc.parallel_loop(0, d, unroll=8)
            def body(dd):
                dd_vec = jnp.full((16,), 0, jnp.int32) + dd
                for cg in range(_CB // 16):
                    vals = plsc.load_gather(tin[slot], [dd_vec, c_iota[cg]])
                    idx = lane_stride + (cg * 16 * d + dd)
                    plsc.store_scatter(dst, [idx], vals)

        fire_in(wid, 0)

        def outer(g2, carry):
            for slot in range(2):
                g = g2 * 2 + slot
                j = g * _NW + wid

                @pl.when(j < n_full)
                def _():
                    pltpu.make_async_copy(
                        wt_hbm.at[:, pl.ds(j * _CB, _CB)], tin[slot], sin[slot]
                    ).wait()
                    nxt = j + _NW

                    @pl.when(nxt < n_full)
                    def _():
                        fire_in(nxt, 1 - slot)

                    @pl.when(g >= 2)
                    def _():
                        pltpu.make_async_copy(
                            tout[slot],
                            stage_hbm.at[pl.ds((j - 2 * _NW) * _CB * d, _CB * d)],
                            sout[slot],
                        ).wait()

                    transpose_block(slot)
                    pltpu.async_copy(
                        tout[slot],
                        stage_hbm.at[pl.ds(j * _CB * d, _CB * d)],
                        sout[slot],
                    )

            return carry

        lax.fori_loop(0, (n_g + 1) // 2, outer, 0)

        n_mine = n_full // _NW + jnp.where(wid < n_full % _NW, 1, 0)
        for slot in range(2):
            last_g = n_mine - 1 - lax.rem(n_mine - 1 - slot, 2)
            j = last_g * _NW + wid

            @pl.when(n_mine > slot)
            def _():
                pltpu.make_async_copy(
                    tout[slot],
                    stage_hbm.at[pl.ds(j * _CB * d, _CB * d)],
                    sout[slot],
                ).wait()

        if tail:
            @pl.when(wid == _NW - 1)
            def _():
                pltpu.sync_copy(wt_hbm.at[:, pl.ds(n_full * _CB, tail)], ttail)

                @plsc.parallel_loop(0, d, unroll=8)
                def body(dd):
                    dd_vec = jnp.full((16,), 0, jnp.int32) + dd
                    for cg in range(tail // 16):
                        vals = plsc.load_gather(ttail, [dd_vec, c_iota[cg]])
                        idx = lane_stride + (cg * 16 * d + dd)
                        plsc.store_scatter(tout0, [idx], vals)
                pltpu.sync_copy(
                    tout0.at[pl.ds(0, tail * d)],
                    stage_hbm.at[pl.ds(n_full * _CB * d, tail * d)],
                )

    return detile_kernel(w_t)


def _sc_gather(index_flat, table, b_total, d):
    b_per_w = b_total // _NW
    n_chunks = b_per_w // _CHUNK
    mesh = plsc.VectorSubcoreMesh(core_axis_name="c", subcore_axis_name="s")

    @functools.partial(
        pl.kernel,
        out_type=jax.ShapeDtypeStruct((b_total, d), jnp.float32),
        mesh=mesh,
        scratch_types=[
            pltpu.VMEM((b_per_w,), jnp.int32),
            pltpu.VMEM((_NBUF, _CHUNK, d), jnp.float32),
        ] + [pltpu.SemaphoreType.DMA] * _NBUF,
        compiler_params=pltpu.CompilerParams(use_tc_tiling_on_sc=False),
    )
    def gather_kernel(idx_hbm, table_hbm, out_hbm, idx_v, rows_v, *sems):
        wid = lax.axis_index("s") * _NC + lax.axis_index("c")
        base = wid * b_per_w
        pltpu.sync_copy(idx_hbm.at[pl.ds(base, b_per_w)], idx_v)

        def fire(i, b):
            off = pl.multiple_of(i * _CHUNK, _CHUNK)
            pltpu.async_copy(
                table_hbm.at[idx_v.at[pl.ds(off, _CHUNK)]],
                rows_v.at[b],
                sems[b],
            )

        for b in range(_NBUF):
            fire(b, b)

        def outer(g, carry):
            for b in range(_NBUF):
                i = g * _NBUF + b
                off = pl.multiple_of(i * _CHUNK, _CHUNK)
                pltpu.make_async_copy(
                    table_hbm.at[idx_v.at[pl.ds(off, _CHUNK)]],
                    rows_v.at[b],
                    sems[b],
                ).wait()
                pltpu.sync_copy(
                    rows_v.at[b], out_hbm.at[pl.ds(base + off, _CHUNK)]
                )
                nxt = i + _NBUF

                @pl.when(nxt < n_chunks)
                def _():
                    fire(nxt, b)

            return carry

        lax.fori_loop(0, n_chunks // _NBUF, outer, 0)

    return gather_kernel(index_flat, table)


@functools.partial(jax.jit, static_argnames=("b_total", "v", "d"))
def _sc_embed(index_flat, emb_weight, *, b_total, v, d):
    stage = _detile_table(emb_weight.T, v, d)
    return _sc_gather(index_flat, stage.reshape(v, d), b_total, d)


def kernel(index, emb_weight):
    b, s = index.shape
    v, d = emb_weight.shape
    out = _sc_embed(index.reshape(-1), emb_weight, b_total=b * s, v=v, d=d)
    return out.reshape(b, s, d)


# speedup vs baseline: 1.4247x; 1.4247x over previous
import functools

import jax
import jax.numpy as jnp
from jax import lax
from jax.experimental import pallas as pl
from jax.experimental.pallas import tpu as pltpu
from jax.experimental.pallas import tpu_sc as plsc

_NC = 2
_NS = 16
_NW = _NC * _NS
_CHUNK = 128
_NBUF = 5


@functools.partial(jax.jit, static_argnames=("b_total", "d"))
def _sc_gather(index_flat, emb_weight, *, b_total, d):
    b_per_w = b_total // _NW
    n_chunks = b_per_w // _CHUNK
    mesh = plsc.VectorSubcoreMesh(core_axis_name="c", subcore_axis_name="s")

    @functools.partial(
        pl.kernel,
        out_type=jax.ShapeDtypeStruct((b_total, d), jnp.float32),
        mesh=mesh,
        scratch_types=[
            pltpu.VMEM((b_per_w,), jnp.int32),
            pltpu.VMEM((_NBUF, _CHUNK, d), jnp.float32),
        ] + [pltpu.SemaphoreType.DMA] * _NBUF,
        compiler_params=pltpu.CompilerParams(use_tc_tiling_on_sc=False),
    )
    def gather_kernel(idx_hbm, table_hbm, out_hbm, idx_v, rows_v, *sems):
        wid = lax.axis_index("s") * _NC + lax.axis_index("c")
        base = wid * b_per_w
        pltpu.sync_copy(idx_hbm.at[pl.ds(base, b_per_w)], idx_v)

        def fire(i, b):
            off = pl.multiple_of(i * _CHUNK, _CHUNK)
            pltpu.async_copy(
                table_hbm.at[idx_v.at[pl.ds(off, _CHUNK)]],
                rows_v.at[b],
                sems[b],
            )

        for b in range(_NBUF):
            fire(b, b)

        def outer(g, carry):
            for b in range(_NBUF):
                i = g * _NBUF + b
                off = pl.multiple_of(i * _CHUNK, _CHUNK)
                pltpu.make_async_copy(
                    table_hbm.at[idx_v.at[pl.ds(off, _CHUNK)]],
                    rows_v.at[b],
                    sems[b],
                ).wait()
                pltpu.sync_copy(
                    rows_v.at[b], out_hbm.at[pl.ds(base + off, _CHUNK)]
                )
                nxt = i + _NBUF

                @pl.when(nxt < n_chunks)
                def _():
                    fire(nxt, b)

            return carry

        lax.fori_loop(0, n_chunks // _NBUF, outer, 0)

    return gather_kernel(index_flat, emb_weight)


def kernel(index, emb_weight):
    b, s = index.shape
    d = emb_weight.shape[1]
    out = _sc_gather(index.reshape(-1), emb_weight, b_total=b * s, d=d)
    return out.reshape(b, s, d)
